# baseline (speedup 1.0000x reference)
import jax
import jax.numpy as jnp
from jax.experimental import pallas as pl
from jax.experimental.pallas import tpu as pltpu

_SINKHORN_ITERS = 5
_TAU = 0.01
_EPS = 1e-6
_LOG2E = 1.4426950408889634
_BS = 2


def _sinkhorn_body(x_ref, o_ref):
    y0s = [x_ref[k] * (_LOG2E / _TAU) for k in range(_BS)]

    rs = []
    for y0 in y0s:
        m = jnp.max(y0, axis=1, keepdims=True)
        s = jnp.sum(jnp.exp2(y0 - m), axis=1, keepdims=True)
        rs.append(m + jnp.log2(s))

    for it in range(_SINKHORN_ITERS):
        last = it == _SINKHORN_ITERS - 1
        cs = []
        for k in range(_BS):
            u = y0s[k] - rs[k]
            m = jnp.max(u, axis=0, keepdims=True)
            e = jnp.exp2(u - m)
            s = jnp.sum(e, axis=0, keepdims=True)
            if last:
                o_ref[k] = e * (1.0 / s) + _EPS
            else:
                cs.append(m + jnp.log2(s))
        if last:
            break
        ts = [y0s[k] - cs[k] for k in range(_BS)]
        ms = [jnp.max(ts[k], axis=1, keepdims=True) for k in range(_BS)]
        rs = []
        for k in range(_BS):
            s = jnp.sum(jnp.exp2(ts[k] - ms[k]), axis=1, keepdims=True)
            rs.append(ms[k] + jnp.log2(s))


def kernel(x):
    b, n, _ = x.shape
    return pl.pallas_call(
        _sinkhorn_body,
        grid=(b // _BS,),
        in_specs=[pl.BlockSpec((_BS, n, n), lambda i: (i, 0, 0))],
        out_specs=pl.BlockSpec((_BS, n, n), lambda i: (i, 0, 0)),
        out_shape=jax.ShapeDtypeStruct(x.shape, x.dtype),
        compiler_params=pltpu.CompilerParams(
            dimension_semantics=("parallel",),
        ),
    )(x)


# speedup vs baseline: 1.0097x; 1.0097x over previous
import jax
import jax.numpy as jnp
from jax.experimental import pallas as pl
from jax.experimental.pallas import tpu as pltpu

_SINKHORN_ITERS = 5
_TAU = 0.01
_EPS = 1e-6
_LOG2E = 1.4426950408889634
_BS = 2


def _sinkhorn_body(x_ref, o_ref):
    y0s = [x_ref[k] * (_LOG2E / _TAU) for k in range(_BS)]

    rs = []
    for y0 in y0s:
        m = jnp.max(y0, axis=1, keepdims=True)
        s = jnp.sum(jnp.exp2(y0 - m), axis=1, keepdims=True)
        rs.append(m + jnp.log2(s))

    for it in range(_SINKHORN_ITERS):
        last = it == _SINKHORN_ITERS - 1
        cs = []
        for k in range(_BS):
            m = jnp.max(y0s[k] - rs[k], axis=0, keepdims=True)
            e = jnp.exp2((y0s[k] - m) - rs[k])
            s = jnp.sum(e, axis=0, keepdims=True)
            if last:
                o_ref[k] = e * (1.0 / s) + _EPS
            else:
                cs.append(m + jnp.log2(s))
        if last:
            break
        rs = []
        for k in range(_BS):
            m = jnp.max(y0s[k] - cs[k], axis=1, keepdims=True)
            s = jnp.sum(
                jnp.exp2((y0s[k] - m) - cs[k]), axis=1, keepdims=True
            )
            rs.append(m + jnp.log2(s))


def kernel(x):
    b, n, _ = x.shape
    return pl.pallas_call(
        _sinkhorn_body,
        grid=(b // _BS,),
        in_specs=[pl.BlockSpec((_BS, n, n), lambda i: (i, 0, 0))],
        out_specs=pl.BlockSpec((_BS, n, n), lambda i: (i, 0, 0)),
        out_shape=jax.ShapeDtypeStruct(x.shape, x.dtype),
        compiler_params=pltpu.CompilerParams(
            dimension_semantics=("parallel",),
        ),
    )(x)


# speedup vs baseline: 1.0419x; 1.0319x over previous
import jax
import jax.numpy as jnp
from jax.experimental import pallas as pl
from jax.experimental.pallas import tpu as pltpu

_SINKHORN_ITERS = 5
_TAU = 0.01
_EPS = 1e-6
_LOG2E = 1.4426950408889634
_BS = 2


def _sinkhorn_body(x_ref, o_ref):
    y0s = [x_ref[k] * (_LOG2E / _TAU) for k in range(_BS)]

    rs = []
    for y0 in y0s:
        m = jnp.max(y0, axis=1, keepdims=True)
        s = jnp.sum(jnp.exp2(y0 - m), axis=1, keepdims=True)
        rs.append(m + jnp.log2(s))

    for it in range(_SINKHORN_ITERS):
        last = it == _SINKHORN_ITERS - 1
        cs = []
        for k in range(_BS):
            u = y0s[k] - rs[k]
            m = jnp.max(u, axis=0, keepdims=True)
            e = jnp.exp2(u - m)
            s = jnp.sum(e, axis=0, keepdims=True)
            if last:
                o_ref[k] = e * (1.0 / s) + _EPS
            else:
                cs.append(m + jnp.log2(s))
        if last:
            break
        rs = []
        for k in range(_BS):
            t = y0s[k] - cs[k]
            m = jnp.max(t, axis=1, keepdims=True)
            s = jnp.sum(jnp.exp2(t - m), axis=1, keepdims=True)
            rs.append(m + jnp.log2(s))


def kernel(x):
    b, n, _ = x.shape
    return pl.pallas_call(
        _sinkhorn_body,
        grid=(b // _BS,),
        in_specs=[pl.BlockSpec((_BS, n, n), lambda i: (i, 0, 0))],
        out_specs=pl.BlockSpec((_BS, n, n), lambda i: (i, 0, 0)),
        out_shape=jax.ShapeDtypeStruct(x.shape, x.dtype),
        compiler_params=pltpu.CompilerParams(
            dimension_semantics=("parallel",),
        ),
    )(x)
